# baseline (speedup 1.0000x reference)
import functools

import jax
import jax.numpy as jnp
from jax import lax
from jax.experimental import pallas as pl
from jax.experimental.pallas import tpu as pltpu
from jax.experimental.pallas import tpu_sc as plsc

B = 64
S = 4
V = 100000
NC = 2
NS = 16
NW = NC * NS
L = 16
NSHARD = 4
NG = 8
GR = 8
SHT = 196
CH = 1280
NFULL = 19
CLAST = 768
TAIL0 = 99968
EPS = 1e-9
LN2 = 0.6931471805599453
SQRT2 = 1.4142135623730951
INF = float("inf")


def _val(lv, uv):
    x = uv + EPS
    bits = lax.bitcast_convert_type(x, jnp.int32)
    e = (bits >> 23) - 127
    m = lax.bitcast_convert_type((bits & 0x007FFFFF) | 0x3F800000, jnp.float32)
    big = m > SQRT2
    m = jnp.where(big, m * 0.5, m)
    e = jnp.where(big, e + 1, e)
    r = (m - 1.0) / (m + 1.0)
    s = r * r
    p = jnp.float32(1.0 / 11.0)
    p = p * s + jnp.float32(1.0 / 9.0)
    p = p * s + jnp.float32(1.0 / 7.0)
    p = p * s + jnp.float32(1.0 / 5.0)
    p = p * s + jnp.float32(1.0 / 3.0)
    p = p * s + jnp.float32(1.0)
    logx = e.astype(jnp.float32) * LN2 + 2.0 * r * p
    return (EPS - logx) * jnp.exp(-lv)


def _argmin_merge(bv, bi, pv, pi):
    upd = (pv < bv) | ((pv == bv) & (pi < bi))
    return jnp.where(upd, pv, bv), jnp.where(upd, pi, bi)


def _sc_partials(l_hbm, u_hbm, tl_hbm, tu_hbm, pv_hbm, pi_hbm,
                 lbuf, ubuf, bvref, biref, ovbuf, oibuf, lsem, usem):
    cid = lax.axis_index("c")
    sid = lax.axis_index("s")
    wid = sid * NC + cid
    g = wid // NSHARD
    v = wid % NSHARD
    soff = jnp.where(v == NSHARD - 1, 585 * 128, v * (SHT * 128))
    lane = lax.iota(jnp.int32, L)

    def lcopy(c, size, slot):
        return pltpu.make_async_copy(
            l_hbm.at[pl.ds(g * GR, GR), pl.ds(0, S),
                     pl.ds(soff + c * CH, size)],
            lbuf.at[slot, pl.ds(0, GR), pl.ds(0, S), pl.ds(0, size)],
            lsem.at[slot],
        )

    def ucopy(c, size, slot):
        return pltpu.make_async_copy(
            u_hbm.at[pl.ds(g * GR, GR), pl.ds(soff + c * CH, size)],
            ubuf.at[slot, pl.ds(0, GR), pl.ds(0, size)],
            usem.at[slot],
        )

    def row_loop(slot, r, base, n_iters):
        def body(i, cr):
            bv = bvref[r, pl.ds(0, L)]
            bi = biref[r, pl.ds(0, L)]
            lv = lbuf[slot, r, S - 1, pl.ds(i * L, L)]
            uv = ubuf[slot, r, pl.ds(i * L, L)]
            val = _val(lv, uv)
            gidx = base + i * L + lane
            upd = val < bv
            bvref[r, pl.ds(0, L)] = jnp.where(upd, val, bv)
            biref[r, pl.ds(0, L)] = jnp.where(upd, gidx, bi)
            return cr

        lax.fori_loop(0, n_iters, body, 0)

    for r in range(GR):
        bvref[r, pl.ds(0, L)] = jnp.full((L,), INF, jnp.float32)
        biref[r, pl.ds(0, L)] = jnp.zeros((L,), jnp.int32)
    lcopy(0, CH, 0).start()
    ucopy(0, CH, 0).start()

    def pair_body(k, cr):
        c0 = 2 * k
        c1 = c0 + 1
        lcopy(c1, CH, 1).start()
        ucopy(c1, CH, 1).start()
        lcopy(c0, CH, 0).wait()
        ucopy(c0, CH, 0).wait()
        for r in range(GR):
            row_loop(0, r, soff + c0 * CH, CH // L)
        lcopy(c1 + 1, CH, 0).start()
        ucopy(c1 + 1, CH, 0).start()
        lcopy(c1, CH, 1).wait()
        ucopy(c1, CH, 1).wait()
        for r in range(GR):
            row_loop(1, r, soff + c1 * CH, CH // L)
        return cr

    lax.fori_loop(0, (NFULL - 1) // 2, pair_body, 0)

    lcopy(NFULL, CLAST, 1).start()
    ucopy(NFULL, CLAST, 1).start()
    lcopy(NFULL - 1, CH, 0).wait()
    ucopy(NFULL - 1, CH, 0).wait()
    for r in range(GR):
        row_loop(0, r, soff + (NFULL - 1) * CH, CH // L)
    lcopy(NFULL, CLAST, 1).wait()
    ucopy(NFULL, CLAST, 1).wait()
    for r in range(GR):
        row_loop(1, r, soff + NFULL * CH, CLAST // L)

    tu = pltpu.make_async_copy(
        tu_hbm.at[pl.ds(g * GR, GR), pl.ds(0, 128)],
        ubuf.at[0, pl.ds(0, GR), pl.ds(0, 128)], usem.at[0])
    tl = pltpu.make_async_copy(
        tl_hbm.at[pl.ds(g * GR, GR), pl.ds(0, 128)],
        ubuf.at[1, pl.ds(0, GR), pl.ds(0, 128)], lsem.at[0])
    tu.start()
    tl.start()
    tu.wait()
    tl.wait()
    f3 = jnp.where(v == NSHARD - 1, jnp.float32(0.0), jnp.float32(INF))
    for r in range(GR):
        for i in range(128 // L):
            bv = bvref[r, pl.ds(0, L)]
            bi = biref[r, pl.ds(0, L)]
            lv = ubuf[1, r, pl.ds(i * L, L)]
            uv = ubuf[0, r, pl.ds(i * L, L)]
            val = _val(lv, uv) + f3
            gidx = TAIL0 + i * L + lane
            upd = val < bv
            bvref[r, pl.ds(0, L)] = jnp.where(upd, val, bv)
            biref[r, pl.ds(0, L)] = jnp.where(upd, gidx, bi)

    for r in range(GR):
        bv = bvref[r, pl.ds(0, L)]
        bi = biref[r, pl.ds(0, L)]
        for st in (8, 4, 2, 1):
            pv = bv.at[lane ^ st].get(mode="promise_in_bounds")
            pi = bi.at[lane ^ st].get(mode="promise_in_bounds")
            bv, bi = _argmin_merge(bv, bi, pv, pi)
        ovbuf[...] = bv
        oibuf[...] = bi
        b = g * GR + r
        off = (v * B + b) * L
        pltpu.make_async_copy(ovbuf, pv_hbm.at[pl.ds(off, L)],
                              lsem.at[0]).start()
        pltpu.make_async_copy(ovbuf, pv_hbm.at[pl.ds(off, L)],
                              lsem.at[0]).wait()
        pltpu.make_async_copy(oibuf, pi_hbm.at[pl.ds(off, L)],
                              usem.at[0]).start()
        pltpu.make_async_copy(oibuf, pi_hbm.at[pl.ds(off, L)],
                              usem.at[0]).wait()


def _sc_merge(pv_hbm, pi_hbm, out_hbm, vbuf, ibuf, obuf, sem):
    cid = lax.axis_index("c")
    sid = lax.axis_index("s")
    wid = sid * NC + cid
    lane = lax.iota(jnp.int32, L)
    vec = jnp.zeros((L,), jnp.int32)
    for r in range(B // NW):
        b = wid * (B // NW) + r
        for v in range(NSHARD):
            off = (v * B + b) * L
            pltpu.make_async_copy(pv_hbm.at[pl.ds(off, L)],
                                  vbuf.at[v, pl.ds(0, L)],
                                  sem.at[0]).start()
            pltpu.make_async_copy(pv_hbm.at[pl.ds(off, L)],
                                  vbuf.at[v, pl.ds(0, L)],
                                  sem.at[0]).wait()
            pltpu.make_async_copy(pi_hbm.at[pl.ds(off, L)],
                                  ibuf.at[v, pl.ds(0, L)],
                                  sem.at[0]).start()
            pltpu.make_async_copy(pi_hbm.at[pl.ds(off, L)],
                                  ibuf.at[v, pl.ds(0, L)],
                                  sem.at[0]).wait()
        mv = vbuf[0, pl.ds(0, L)]
        mi = ibuf[0, pl.ds(0, L)]
        for v in range(1, NSHARD):
            mv, mi = _argmin_merge(mv, mi, vbuf[v, pl.ds(0, L)],
                                   ibuf[v, pl.ds(0, L)])
        vec = jnp.where(lane == r * (L * NW // B), mi, vec)
    obuf[...] = vec
    pltpu.make_async_copy(obuf, out_hbm.at[wid], sem.at[0]).start()
    pltpu.make_async_copy(obuf, out_hbm.at[wid], sem.at[0]).wait()


def kernel(logits, u):
    tail_l = jnp.pad(logits[:, S - 1, TAIL0:], ((0, 0), (0, 96)),
                     constant_values=-jnp.inf)
    tail_u = jnp.pad(u[:, TAIL0:], ((0, 0), (0, 96)), constant_values=0.5)
    mesh = plsc.VectorSubcoreMesh(core_axis_name="c", subcore_axis_name="s")
    k1 = functools.partial(
        pl.kernel,
        out_type=(
            jax.ShapeDtypeStruct((NSHARD * B * L,), jnp.float32),
            jax.ShapeDtypeStruct((NSHARD * B * L,), jnp.int32),
        ),
        mesh=mesh,
        scratch_types=[
            pltpu.VMEM((2, GR, S, CH), jnp.float32),
            pltpu.VMEM((2, GR, CH), jnp.float32),
            pltpu.VMEM((GR, L), jnp.float32),
            pltpu.VMEM((GR, L), jnp.int32),
            pltpu.VMEM((L,), jnp.float32),
            pltpu.VMEM((L,), jnp.int32),
            pltpu.SemaphoreType.DMA((2,)),
            pltpu.SemaphoreType.DMA((2,)),
        ],
    )(_sc_partials)
    pvals, pidx = k1(logits, u, tail_l, tail_u)
    k2 = functools.partial(
        pl.kernel,
        out_type=jax.ShapeDtypeStruct((NW, L), jnp.int32),
        mesh=mesh,
        scratch_types=[
            pltpu.VMEM((NSHARD, L), jnp.float32),
            pltpu.VMEM((NSHARD, L), jnp.int32),
            pltpu.VMEM((L,), jnp.int32),
            pltpu.SemaphoreType.DMA((2,)),
        ],
    )(_sc_merge)
    out2d = k2(pvals, pidx)
    return out2d[:, :: (L * NW // B)].reshape(B)


# speedup vs baseline: 11.0298x; 11.0298x over previous
import jax
import jax.numpy as jnp
from jax.experimental import pallas as pl
from jax.experimental.pallas import tpu as pltpu

B = 64
S = 4
V = 100000
VB = 11264
NBLK = (V + VB - 1) // VB
EPS = 1e-9


def _tc_body(l_ref, u_ref, out_ref, rm_ref, ra_ref):
    j = pl.program_id(0)

    @pl.when(j == 0)
    def _init():
        rm_ref[...] = jnp.full((B, 128), -jnp.inf, jnp.float32)
        ra_ref[...] = jnp.zeros((B, 128), jnp.int32)

    l = l_ref[:, S - 1, :]
    u = u_ref[...]
    g = -jnp.log(-jnp.log(u + EPS) + EPS)
    val = l + g
    col = j * VB + jax.lax.broadcasted_iota(jnp.int32, (B, VB), 1)
    val = jnp.where(col < V, val, -jnp.inf)
    bm = jnp.max(val, axis=1, keepdims=True)
    cand = jnp.where(val == bm, col, jnp.int32(2**31 - 1))
    ba = jnp.min(cand, axis=1, keepdims=True)
    rm = rm_ref[...]
    upd = bm > rm
    ra_ref[...] = jnp.where(upd, ba, ra_ref[...])
    rm_ref[...] = jnp.where(upd, bm, rm)

    @pl.when(j == NBLK - 1)
    def _fin():
        out_ref[...] = ra_ref[...]


def kernel(logits, u):
    out = pl.pallas_call(
        _tc_body,
        grid=(NBLK,),
        in_specs=[
            pl.BlockSpec((B, S, VB), lambda j: (0, 0, j)),
            pl.BlockSpec((B, VB), lambda j: (0, j)),
        ],
        out_specs=pl.BlockSpec((B, 128), lambda j: (0, 0)),
        out_shape=jax.ShapeDtypeStruct((B, 128), jnp.int32),
        scratch_shapes=[
            pltpu.VMEM((B, 128), jnp.float32),
            pltpu.VMEM((B, 128), jnp.int32),
        ],
    )(logits, u)
    return out[:, 0]


# speedup vs baseline: 11.3871x; 1.0324x over previous
import jax
import jax.numpy as jnp
from jax.experimental import pallas as pl
from jax.experimental.pallas import tpu as pltpu

B = 64
S = 4
V = 100000
VB = 11264
NBLK = (V + VB - 1) // VB
EPS = 1e-9


def _tc_body(l_ref, u_ref, out_ref, rm_ref, ra_ref):
    j = pl.program_id(0)

    @pl.when(j == 0)
    def _init():
        rm_ref[...] = jnp.full((B, 128), -jnp.inf, jnp.float32)
        ra_ref[...] = jnp.zeros((B, 128), jnp.int32)

    l = l_ref[:, S - 1, :]
    u = u_ref[...]
    g = -jnp.log(-jnp.log(u + EPS) + EPS)
    val = l + g
    col = j * VB + jax.lax.broadcasted_iota(jnp.int32, (B, VB), 1)
    val = jnp.where(col < V, val, -jnp.inf)
    bm = jnp.max(val, axis=1, keepdims=True)
    cand = jnp.where(val == bm, col, jnp.int32(2**31 - 1))
    ba = jnp.min(cand, axis=1, keepdims=True)
    rm = rm_ref[...]
    upd = bm > rm
    ra_ref[...] = jnp.where(upd, ba, ra_ref[...])
    rm_ref[...] = jnp.where(upd, bm, rm)

    @pl.when(j == NBLK - 1)
    def _fin():
        out_ref[...] = ra_ref[:, 0]


def kernel(logits, u):
    out = pl.pallas_call(
        _tc_body,
        grid=(NBLK,),
        in_specs=[
            pl.BlockSpec((B, S, VB), lambda j: (0, 0, j)),
            pl.BlockSpec((B, VB), lambda j: (0, j)),
        ],
        out_specs=pl.BlockSpec((B,), lambda j: (0,)),
        out_shape=jax.ShapeDtypeStruct((B,), jnp.int32),
        scratch_shapes=[
            pltpu.VMEM((B, 128), jnp.float32),
            pltpu.VMEM((B, 128), jnp.int32),
        ],
    )(logits, u)
    return out
